# baseline (speedup 1.0000x reference)
import jax
import jax.numpy as jnp
from jax.experimental import pallas as pl
from jax.experimental.pallas import tpu as pltpu


def _adj_kernel(z1_ref, z2_ref, rk_ref, adj_ref, rk2_ref, z1c_ref, z2c_ref):
    logits = jax.lax.dot_general(
        z1_ref[0], z2_ref[0], (((1,), (1,)), ((), ())),
        preferred_element_type=jnp.float32,
    )
    adj_ref[0] = 0.5 * jnp.tanh(0.5 * logits) + 0.5
    rk2_ref[...] = jax.nn.sigmoid(rk_ref[...])
    z1c_ref[...] = z1_ref[...]
    z2c_ref[...] = z2_ref[...]


@jax.jit
def kernel(z1, z2, rk_lgt):
    B, N, Z = z1.shape
    BM = 1024
    grid = (B, N // BM)

    adj, rk2, z1c, z2c = pl.pallas_call(
        _adj_kernel,
        grid=grid,
        in_specs=[
            pl.BlockSpec((1, BM, Z), lambda b, i: (b, i, 0)),
            pl.BlockSpec((1, N, Z), lambda b, i: (b, 0, 0)),
            pl.BlockSpec((1, Z), lambda b, i: (0, 0)),
        ],
        out_specs=[
            pl.BlockSpec((1, BM, N), lambda b, i: (b, i, 0)),
            pl.BlockSpec((1, Z), lambda b, i: (0, 0)),
            pl.BlockSpec((1, BM, Z), lambda b, i: (b, i, 0)),
            pl.BlockSpec((1, N, Z), lambda b, i: (b, 0, 0)),
        ],
        out_shape=[
            jax.ShapeDtypeStruct((B, N, N), jnp.float32),
            jax.ShapeDtypeStruct(rk_lgt.shape, jnp.float32),
            jax.ShapeDtypeStruct((B, N, Z), jnp.float32),
            jax.ShapeDtypeStruct((B, N, Z), jnp.float32),
        ],
        compiler_params=pltpu.CompilerParams(
            dimension_semantics=("parallel", "arbitrary"),
        ),
    )(z1, z2, rk_lgt)

    return (adj, z1c, z2c, rk2)


# speedup vs baseline: 1.1289x; 1.1289x over previous
import jax
import jax.numpy as jnp
from jax.experimental import pallas as pl
from jax.experimental.pallas import tpu as pltpu


def _adj_kernel(z1_ref, z2_ref, rk_ref, adj_ref, rk2_ref):
    logits = jax.lax.dot_general(
        z1_ref[0], z2_ref[0], (((1,), (1,)), ((), ())),
        preferred_element_type=jnp.float32,
    )
    adj_ref[0] = 0.5 * jnp.tanh(0.5 * logits) + 0.5
    rk2_ref[...] = jax.nn.sigmoid(rk_ref[...])


@jax.jit
def kernel(z1, z2, rk_lgt):
    B, N, Z = z1.shape
    BM = 1024
    grid = (B, N // BM)

    adj, rk2 = pl.pallas_call(
        _adj_kernel,
        grid=grid,
        in_specs=[
            pl.BlockSpec((1, BM, Z), lambda b, i: (b, i, 0)),
            pl.BlockSpec((1, N, Z), lambda b, i: (b, 0, 0)),
            pl.BlockSpec((1, Z), lambda b, i: (0, 0)),
        ],
        out_specs=[
            pl.BlockSpec((1, BM, N), lambda b, i: (b, i, 0)),
            pl.BlockSpec((1, Z), lambda b, i: (0, 0)),
        ],
        out_shape=[
            jax.ShapeDtypeStruct((B, N, N), jnp.float32),
            jax.ShapeDtypeStruct(rk_lgt.shape, jnp.float32),
        ],
        compiler_params=pltpu.CompilerParams(
            dimension_semantics=("parallel", "arbitrary"),
        ),
    )(z1, z2, rk_lgt)

    return (adj, z1, z2, rk2)


# speedup vs baseline: 1.1337x; 1.0043x over previous
import jax
import jax.numpy as jnp
from jax.experimental import pallas as pl
from jax.experimental.pallas import tpu as pltpu


def _adj_kernel(z1_ref, z2_ref, rk_ref, adj_ref, rk2_ref):
    logits = jax.lax.dot_general(
        z1_ref[0], z2_ref[0], (((1,), (1,)), ((), ())),
        preferred_element_type=jnp.float32,
    )
    adj_ref[0] = 0.5 * jnp.tanh(0.5 * logits) + 0.5
    rk2_ref[...] = jax.nn.sigmoid(rk_ref[...])


@jax.jit
def kernel(z1, z2, rk_lgt):
    B, N, Z = z1.shape
    BM = 512
    grid = (B, N // BM)

    adj, rk2 = pl.pallas_call(
        _adj_kernel,
        grid=grid,
        in_specs=[
            pl.BlockSpec((1, BM, Z), lambda b, i: (b, i, 0)),
            pl.BlockSpec((1, N, Z), lambda b, i: (b, 0, 0)),
            pl.BlockSpec((1, Z), lambda b, i: (0, 0)),
        ],
        out_specs=[
            pl.BlockSpec((1, BM, N), lambda b, i: (b, i, 0)),
            pl.BlockSpec((1, Z), lambda b, i: (0, 0)),
        ],
        out_shape=[
            jax.ShapeDtypeStruct((B, N, N), jnp.float32),
            jax.ShapeDtypeStruct(rk_lgt.shape, jnp.float32),
        ],
        compiler_params=pltpu.CompilerParams(
            dimension_semantics=("parallel", "arbitrary"),
        ),
    )(z1, z2, rk_lgt)

    return (adj, z1, z2, rk2)
